# Initial kernel scaffold; baseline (speedup 1.0000x reference)
import jax
import jax.numpy as jnp
from jax.experimental import pallas as pl


def kernel(x, edge_index, batch, Wl1, bl1, Wr1, Wl2, bl2, Wr2, C1W, C1b, C2W, C2b, C3W, C3b):
    raise NotImplementedError("write your pallas kernel here")


# speedup vs baseline: 2.7587x; 2.7587x over previous
import functools

import jax
import jax.numpy as jnp
from jax import lax
from jax.experimental import pallas as pl
from jax.experimental.pallas import tpu as pltpu
from jax.experimental.pallas import tpu_sc as plsc

N = 10000
D = 256
E = 160000
G = 64
HALF = 128

NP = 10240
EP = 163840
NCHUNK = EP // 128
ROWS_PER_TILE = NP // 16
CHUNKS_PER_TILE = NCHUNK // 16
SUPERS = CHUNKS_PER_TILE // 8
NSTRIPE = ROWS_PER_TILE // 128

B = 512
NB = NP // B

_MESH = plsc.VectorSubcoreMesh(
    core_axis_name="c", subcore_axis_name="s", num_cores=2, num_subcores=16)


def _sc_agg_body(h2, srcx, dst2, zrow, agg_out, sidx, didx, rows, agg_sh, sem):
  c = lax.axis_index("c")
  s = lax.axis_index("s")
  rbase = s * ROWS_PER_TILE

  pltpu.sync_copy(zrow, rows)
  for m in range(NSTRIPE):
    pltpu.sync_copy(rows, agg_sh.at[pl.ds(rbase + m * 128, 128)])
  plsc.subcore_barrier()

  cb0 = s * CHUNKS_PER_TILE
  sbase = c * NCHUNK + cb0

  def super_body(k):
    pltpu.sync_copy(srcx.at[pl.ds(sbase + k * 8, 8)], sidx)
    pltpu.sync_copy(dst2.at[pl.ds(cb0 + k * 8, 8)], didx)
    for j in range(8):
      pltpu.async_copy(h2.at[sidx.at[j]], rows, sem).wait()
      pltpu.sync_copy(rows, agg_sh.at[didx.at[j]], add=True)

  pl.loop(0, SUPERS)(super_body)

  plsc.subcore_barrier()
  for m in range(NSTRIPE):
    pltpu.sync_copy(agg_sh.at[pl.ds(rbase + m * 128, 128)], rows)
    pltpu.sync_copy(rows, agg_out.at[pl.ds(c * NP + rbase + m * 128, 128)])


_sc_agg = pl.kernel(
    _sc_agg_body,
    out_type=jax.ShapeDtypeStruct((2 * NP, HALF), jnp.float32),
    mesh=_MESH,
    scratch_types=[
        pltpu.VMEM((8, 128), jnp.int32),
        pltpu.VMEM((8, 128), jnp.int32),
        pltpu.VMEM((128, HALF), jnp.float32),
        pltpu.VMEM_SHARED((NP, HALF), jnp.float32),
        pltpu.SemaphoreType.DMA,
    ],
)


def _sc_deg_body(dst2, zrow, onesr, deg_out, didx, onesv, buf, deg_sh):
  c = lax.axis_index("c")
  s = lax.axis_index("s")
  rbase = s * ROWS_PER_TILE

  @pl.when(c == 0)
  def _():
    pltpu.sync_copy(onesr, onesv)
    pltpu.sync_copy(zrow, buf)
    for m in range(NSTRIPE):
      pltpu.sync_copy(buf, deg_sh.at[pl.ds(rbase + m * 128, 128)])
  plsc.subcore_barrier()

  cb0 = s * CHUNKS_PER_TILE

  @pl.when(c == 0)
  def _():
    def super_body(k):
      pltpu.sync_copy(dst2.at[pl.ds(cb0 + k * 8, 8)], didx)
      for j in range(8):
        pltpu.sync_copy(onesv, deg_sh.at[didx.at[j]], add=True)
    pl.loop(0, SUPERS)(super_body)

  plsc.subcore_barrier()

  @pl.when(c == 0)
  def _():
    for m in range(NSTRIPE):
      pltpu.sync_copy(deg_sh.at[pl.ds(rbase + m * 128, 128)], buf)
      pltpu.sync_copy(buf, deg_out.at[pl.ds(rbase + m * 128, 128)])


_sc_deg = pl.kernel(
    _sc_deg_body,
    out_type=jax.ShapeDtypeStruct((NP, HALF), jnp.float32),
    mesh=_MESH,
    scratch_types=[
        pltpu.VMEM((8, 128), jnp.int32),
        pltpu.VMEM((128, HALF), jnp.float32),
        pltpu.VMEM((128, HALF), jnp.float32),
        pltpu.VMEM_SHARED((NP, HALF), jnp.float32),
    ],
)


def _t1_body(agg_ref, deg_ref, x_ref, wl0_ref, wl1_ref, wr_ref, b_ref,
             out_ref):
  r = 1.0 / jnp.maximum(deg_ref[...], 1.0)
  a0 = agg_ref[0] * r
  a1 = agg_ref[1] * r
  h = jnp.dot(a0, wl0_ref[...], preferred_element_type=jnp.float32)
  h = h + jnp.dot(a1, wl1_ref[...], preferred_element_type=jnp.float32)
  h = h + jnp.dot(x_ref[...], wr_ref[...], preferred_element_type=jnp.float32)
  h = h + b_ref[0:1]
  h = jnp.maximum(h, 0.0)
  out_ref[0] = h[:, :HALF]
  out_ref[1] = h[:, HALF:]


_t1 = pl.pallas_call(
    _t1_body,
    grid=(NB,),
    in_specs=[
        pl.BlockSpec((2, B, HALF), lambda i: (0, i, 0)),
        pl.BlockSpec((B, HALF), lambda i: (i, 0)),
        pl.BlockSpec((B, D), lambda i: (i, 0)),
        pl.BlockSpec((HALF, D), lambda i: (0, 0)),
        pl.BlockSpec((HALF, D), lambda i: (0, 0)),
        pl.BlockSpec((D, D), lambda i: (0, 0)),
        pl.BlockSpec((8, D), lambda i: (0, 0)),
    ],
    out_specs=pl.BlockSpec((2, B, HALF), lambda i: (0, i, 0)),
    out_shape=jax.ShapeDtypeStruct((2, NP, HALF), jnp.float32),
)


def _t2_body(agg_ref, deg_ref, h1_ref, wl0_ref, wl1_ref, wr0_ref, wr1_ref,
             b_ref, batch_ref, c1w_ref, c1b_ref, c2w_ref, c2b_ref, c3w_ref,
             c3b_ref, out_ref, psum, cnt):
  i = pl.program_id(0)

  @pl.when(i == 0)
  def _():
    psum[...] = jnp.zeros_like(psum)
    cnt[...] = jnp.zeros_like(cnt)

  r = 1.0 / jnp.maximum(deg_ref[...], 1.0)
  a0 = agg_ref[0] * r
  a1 = agg_ref[1] * r
  h = jnp.dot(a0, wl0_ref[...], preferred_element_type=jnp.float32)
  h = h + jnp.dot(a1, wl1_ref[...], preferred_element_type=jnp.float32)
  h = h + jnp.dot(h1_ref[0], wr0_ref[...], preferred_element_type=jnp.float32)
  h = h + jnp.dot(h1_ref[1], wr1_ref[...], preferred_element_type=jnp.float32)
  h = h + b_ref[0:1]
  h2 = jnp.maximum(h, 0.0)

  b = batch_ref[0]
  gid = lax.broadcasted_iota(jnp.int32, (G, B), 0)
  rowid = lax.broadcasted_iota(jnp.int32, (G, B), 1) + i * B
  mask = jnp.where((b == gid) & (rowid < N), 1.0, 0.0)
  psum[...] += jnp.dot(mask, h2, preferred_element_type=jnp.float32)
  cnt[...] += jnp.sum(mask, axis=1, keepdims=True)

  @pl.when(i == NB - 1)
  def _():
    cw = cnt[:, 0:1]
    pooled = psum[...] / jnp.maximum(cw, 1.0)
    z = jnp.dot(pooled, c1w_ref[...], preferred_element_type=jnp.float32)
    z = jnp.maximum(z + c1b_ref[0:1], 0.0)
    z = jnp.dot(z, c2w_ref[...], preferred_element_type=jnp.float32)
    z = jnp.maximum(z + c2b_ref[0:1], 0.0)
    o = jnp.dot(z, c3w_ref[...], preferred_element_type=jnp.float32)
    out_ref[...] = o + c3b_ref[0:1]


_t2 = pl.pallas_call(
    _t2_body,
    grid=(NB,),
    in_specs=[
        pl.BlockSpec((2, B, HALF), lambda i: (0, i, 0)),
        pl.BlockSpec((B, HALF), lambda i: (i, 0)),
        pl.BlockSpec((2, B, HALF), lambda i: (0, i, 0)),
        pl.BlockSpec((HALF, D), lambda i: (0, 0)),
        pl.BlockSpec((HALF, D), lambda i: (0, 0)),
        pl.BlockSpec((HALF, D), lambda i: (0, 0)),
        pl.BlockSpec((HALF, D), lambda i: (0, 0)),
        pl.BlockSpec((8, D), lambda i: (0, 0)),
        pl.BlockSpec((1, 1, B), lambda i: (i, 0, 0)),
        pl.BlockSpec((D, HALF), lambda i: (0, 0)),
        pl.BlockSpec((8, HALF), lambda i: (0, 0)),
        pl.BlockSpec((HALF, HALF), lambda i: (0, 0)),
        pl.BlockSpec((8, HALF), lambda i: (0, 0)),
        pl.BlockSpec((HALF, HALF), lambda i: (0, 0)),
        pl.BlockSpec((8, HALF), lambda i: (0, 0)),
    ],
    out_specs=pl.BlockSpec((G, HALF), lambda i: (0, 0)),
    out_shape=jax.ShapeDtypeStruct((G, HALF), jnp.float32),
    scratch_shapes=[
        pltpu.VMEM((G, D), jnp.float32),
        pltpu.VMEM((G, HALF), jnp.float32),
    ],
)


def kernel(x, edge_index, batch, Wl1, bl1, Wr1, Wl2, bl2, Wr2,
           C1W, C1b, C2W, C2b, C3W, C3b):
  f32 = jnp.float32
  src = edge_index[0].astype(jnp.int32)
  dst = edge_index[1].astype(jnp.int32)

  pad_e = EP - E
  srcp = jnp.concatenate([src, jnp.zeros((pad_e,), jnp.int32)])
  srcx = jnp.concatenate([srcp, srcp + NP]).reshape(2 * NCHUNK, 128)
  dst2 = jnp.concatenate([dst, jnp.full((pad_e,), N, jnp.int32)]).reshape(NCHUNK, 128)

  xp = jnp.pad(x, ((0, NP - N), (0, 0)))
  x2 = jnp.concatenate([xp[:, :HALF], xp[:, HALF:]], axis=0)

  zrow = jnp.zeros((128, HALF), f32)
  onesr = jnp.ones((128, HALF), f32)

  deg = _sc_deg(dst2, zrow, onesr)
  agg1 = _sc_agg(x2, srcx, dst2, zrow).reshape(2, NP, HALF)

  wl1t = Wl1.T
  bl1b = jnp.broadcast_to(bl1[None, :], (8, D))
  h1 = _t1(agg1, deg, xp, wl1t[:HALF], wl1t[HALF:], Wr1.T, bl1b)

  agg2 = _sc_agg(h1.reshape(2 * NP, HALF), srcx, dst2, zrow).reshape(2, NP, HALF)

  wl2t = Wl2.T
  wr2t = Wr2.T
  bl2b = jnp.broadcast_to(bl2[None, :], (8, D))
  batch3 = jnp.concatenate([batch.astype(jnp.int32),
                            jnp.full((NP - N,), G, jnp.int32)]).reshape(NB, 1, B)
  c1wt = C1W.T
  c1bb = jnp.broadcast_to(C1b[None, :], (8, HALF))
  c2wt = jnp.zeros((HALF, HALF), f32).at[:, :64].set(C2W.T)
  c2bb = jnp.broadcast_to(jnp.zeros((HALF,), f32).at[:64].set(C2b)[None, :], (8, HALF))
  c3wt = jnp.zeros((HALF, HALF), f32).at[:64, 0].set(C3W[0])
  c3bb = jnp.broadcast_to(jnp.zeros((HALF,), f32).at[0].set(C3b[0])[None, :], (8, HALF))

  out128 = _t2(agg2, deg, h1, wl2t[:HALF], wl2t[HALF:],
               wr2t[:HALF], wr2t[HALF:], bl2b, batch3,
               c1wt, c1bb, c2wt, c2bb, c3wt, c3bb)
  return out128[:, :1]


# speedup vs baseline: 2.9626x; 1.0739x over previous
import functools

import jax
import jax.numpy as jnp
from jax import lax
from jax.experimental import pallas as pl
from jax.experimental.pallas import tpu as pltpu
from jax.experimental.pallas import tpu_sc as plsc

N = 10000
D = 256
E = 160000
G = 64
HALF = 128

NP = 10240
EP = 163840
NCHUNK = EP // 128
ROWS_PER_TILE = NP // 16
CHUNKS_PER_TILE = NCHUNK // 16
SUPERS = CHUNKS_PER_TILE // 8
NSTRIPE = ROWS_PER_TILE // 128

B = 512
NB = NP // B

_MESH = plsc.VectorSubcoreMesh(
    core_axis_name="c", subcore_axis_name="s", num_cores=2, num_subcores=16)


def _sc_agg_body(h2, srcx, dst2, zrow, agg_out, sidx, didx, rows0, rows1,
                 agg_sh, gsem0, gsem1, ssem0, ssem1):
  c = lax.axis_index("c")
  s = lax.axis_index("s")
  rbase = s * ROWS_PER_TILE

  pltpu.sync_copy(zrow, rows0)
  for m in range(NSTRIPE):
    pltpu.sync_copy(rows0, agg_sh.at[pl.ds(rbase + m * 128, 128)])
  plsc.subcore_barrier()

  cb0 = s * CHUNKS_PER_TILE
  sbase = c * NCHUNK + cb0

  rows = (rows0, rows1)
  gsem = (gsem0, gsem1)
  ssem = (ssem0, ssem1)

  def super_body(k):
    pltpu.sync_copy(srcx.at[pl.ds(sbase + k * 8, 8)], sidx)
    pltpu.sync_copy(dst2.at[pl.ds(cb0 + k * 8, 8)], didx)
    pltpu.async_copy(h2.at[sidx.at[0]], rows0, gsem0)
    pltpu.async_copy(h2.at[sidx.at[1]], rows1, gsem1)
    for p in range(3):
      for u in range(2):
        ch = 2 * p + u
        pltpu.make_async_copy(h2.at[sidx.at[ch]], rows[u], gsem[u]).wait()
        pltpu.async_copy(rows[u], agg_sh.at[didx.at[ch]], ssem[u], add=True)
      for u in range(2):
        ch = 2 * p + u
        pltpu.make_async_copy(rows[u], agg_sh.at[didx.at[ch]], ssem[u]).wait()
        pltpu.async_copy(h2.at[sidx.at[ch + 2]], rows[u], gsem[u])
    for u in range(2):
      ch = 6 + u
      pltpu.make_async_copy(h2.at[sidx.at[ch]], rows[u], gsem[u]).wait()
      pltpu.async_copy(rows[u], agg_sh.at[didx.at[ch]], ssem[u], add=True)
    for u in range(2):
      ch = 6 + u
      pltpu.make_async_copy(rows[u], agg_sh.at[didx.at[ch]], ssem[u]).wait()

  pl.loop(0, SUPERS)(super_body)

  plsc.subcore_barrier()
  for m in range(NSTRIPE):
    pltpu.sync_copy(agg_sh.at[pl.ds(rbase + m * 128, 128)], rows0)
    pltpu.sync_copy(rows0, agg_out.at[pl.ds(c * NP + rbase + m * 128, 128)])


_sc_agg = pl.kernel(
    _sc_agg_body,
    out_type=jax.ShapeDtypeStruct((2 * NP, HALF), jnp.float32),
    mesh=_MESH,
    scratch_types=[
        pltpu.VMEM((8, 128), jnp.int32),
        pltpu.VMEM((8, 128), jnp.int32),
        pltpu.VMEM((128, HALF), jnp.float32),
        pltpu.VMEM((128, HALF), jnp.float32),
        pltpu.VMEM_SHARED((NP, HALF), jnp.float32),
        pltpu.SemaphoreType.DMA,
        pltpu.SemaphoreType.DMA,
        pltpu.SemaphoreType.DMA,
        pltpu.SemaphoreType.DMA,
    ],
)


def _sc_deg_body(dst2, zrow, onesr, deg_out, didx, onesv, buf, deg_sh):
  c = lax.axis_index("c")
  s = lax.axis_index("s")
  rbase = s * ROWS_PER_TILE

  @pl.when(c == 0)
  def _():
    pltpu.sync_copy(onesr, onesv)
    pltpu.sync_copy(zrow, buf)
    for m in range(NSTRIPE):
      pltpu.sync_copy(buf, deg_sh.at[pl.ds(rbase + m * 128, 128)])
  plsc.subcore_barrier()

  cb0 = s * CHUNKS_PER_TILE

  @pl.when(c == 0)
  def _():
    def super_body(k):
      pltpu.sync_copy(dst2.at[pl.ds(cb0 + k * 8, 8)], didx)
      for j in range(8):
        pltpu.sync_copy(onesv, deg_sh.at[didx.at[j]], add=True)
    pl.loop(0, SUPERS)(super_body)

  plsc.subcore_barrier()

  @pl.when(c == 0)
  def _():
    for m in range(NSTRIPE):
      pltpu.sync_copy(deg_sh.at[pl.ds(rbase + m * 128, 128)], buf)
      pltpu.sync_copy(buf, deg_out.at[pl.ds(rbase + m * 128, 128)])


_sc_deg = pl.kernel(
    _sc_deg_body,
    out_type=jax.ShapeDtypeStruct((NP, HALF), jnp.float32),
    mesh=_MESH,
    scratch_types=[
        pltpu.VMEM((8, 128), jnp.int32),
        pltpu.VMEM((128, HALF), jnp.float32),
        pltpu.VMEM((128, HALF), jnp.float32),
        pltpu.VMEM_SHARED((NP, HALF), jnp.float32),
    ],
)


def _t1_body(agg_ref, deg_ref, x_ref, wl0_ref, wl1_ref, wr_ref, b_ref,
             out_ref):
  r = 1.0 / jnp.maximum(deg_ref[...], 1.0)
  a0 = agg_ref[0] * r
  a1 = agg_ref[1] * r
  h = jnp.dot(a0, wl0_ref[...], preferred_element_type=jnp.float32)
  h = h + jnp.dot(a1, wl1_ref[...], preferred_element_type=jnp.float32)
  h = h + jnp.dot(x_ref[...], wr_ref[...], preferred_element_type=jnp.float32)
  h = h + b_ref[0:1]
  h = jnp.maximum(h, 0.0)
  out_ref[0] = h[:, :HALF]
  out_ref[1] = h[:, HALF:]


_t1 = pl.pallas_call(
    _t1_body,
    grid=(NB,),
    in_specs=[
        pl.BlockSpec((2, B, HALF), lambda i: (0, i, 0)),
        pl.BlockSpec((B, HALF), lambda i: (i, 0)),
        pl.BlockSpec((B, D), lambda i: (i, 0)),
        pl.BlockSpec((HALF, D), lambda i: (0, 0)),
        pl.BlockSpec((HALF, D), lambda i: (0, 0)),
        pl.BlockSpec((D, D), lambda i: (0, 0)),
        pl.BlockSpec((8, D), lambda i: (0, 0)),
    ],
    out_specs=pl.BlockSpec((2, B, HALF), lambda i: (0, i, 0)),
    out_shape=jax.ShapeDtypeStruct((2, NP, HALF), jnp.float32),
)


def _t2_body(agg_ref, deg_ref, h1_ref, wl0_ref, wl1_ref, wr0_ref, wr1_ref,
             b_ref, batch_ref, c1w_ref, c1b_ref, c2w_ref, c2b_ref, c3w_ref,
             c3b_ref, out_ref, psum, cnt):
  i = pl.program_id(0)

  @pl.when(i == 0)
  def _():
    psum[...] = jnp.zeros_like(psum)
    cnt[...] = jnp.zeros_like(cnt)

  r = 1.0 / jnp.maximum(deg_ref[...], 1.0)
  a0 = agg_ref[0] * r
  a1 = agg_ref[1] * r
  h = jnp.dot(a0, wl0_ref[...], preferred_element_type=jnp.float32)
  h = h + jnp.dot(a1, wl1_ref[...], preferred_element_type=jnp.float32)
  h = h + jnp.dot(h1_ref[0], wr0_ref[...], preferred_element_type=jnp.float32)
  h = h + jnp.dot(h1_ref[1], wr1_ref[...], preferred_element_type=jnp.float32)
  h = h + b_ref[0:1]
  h2 = jnp.maximum(h, 0.0)

  b = batch_ref[0]
  gid = lax.broadcasted_iota(jnp.int32, (G, B), 0)
  rowid = lax.broadcasted_iota(jnp.int32, (G, B), 1) + i * B
  mask = jnp.where((b == gid) & (rowid < N), 1.0, 0.0)
  psum[...] += jnp.dot(mask, h2, preferred_element_type=jnp.float32)
  cnt[...] += jnp.sum(mask, axis=1, keepdims=True)

  @pl.when(i == NB - 1)
  def _():
    cw = cnt[:, 0:1]
    pooled = psum[...] / jnp.maximum(cw, 1.0)
    z = jnp.dot(pooled, c1w_ref[...], preferred_element_type=jnp.float32)
    z = jnp.maximum(z + c1b_ref[0:1], 0.0)
    z = jnp.dot(z, c2w_ref[...], preferred_element_type=jnp.float32)
    z = jnp.maximum(z + c2b_ref[0:1], 0.0)
    o = jnp.dot(z, c3w_ref[...], preferred_element_type=jnp.float32)
    out_ref[...] = o + c3b_ref[0:1]


_t2 = pl.pallas_call(
    _t2_body,
    grid=(NB,),
    in_specs=[
        pl.BlockSpec((2, B, HALF), lambda i: (0, i, 0)),
        pl.BlockSpec((B, HALF), lambda i: (i, 0)),
        pl.BlockSpec((2, B, HALF), lambda i: (0, i, 0)),
        pl.BlockSpec((HALF, D), lambda i: (0, 0)),
        pl.BlockSpec((HALF, D), lambda i: (0, 0)),
        pl.BlockSpec((HALF, D), lambda i: (0, 0)),
        pl.BlockSpec((HALF, D), lambda i: (0, 0)),
        pl.BlockSpec((8, D), lambda i: (0, 0)),
        pl.BlockSpec((1, 1, B), lambda i: (i, 0, 0)),
        pl.BlockSpec((D, HALF), lambda i: (0, 0)),
        pl.BlockSpec((8, HALF), lambda i: (0, 0)),
        pl.BlockSpec((HALF, HALF), lambda i: (0, 0)),
        pl.BlockSpec((8, HALF), lambda i: (0, 0)),
        pl.BlockSpec((HALF, HALF), lambda i: (0, 0)),
        pl.BlockSpec((8, HALF), lambda i: (0, 0)),
    ],
    out_specs=pl.BlockSpec((G, HALF), lambda i: (0, 0)),
    out_shape=jax.ShapeDtypeStruct((G, HALF), jnp.float32),
    scratch_shapes=[
        pltpu.VMEM((G, D), jnp.float32),
        pltpu.VMEM((G, HALF), jnp.float32),
    ],
)


def kernel(x, edge_index, batch, Wl1, bl1, Wr1, Wl2, bl2, Wr2,
           C1W, C1b, C2W, C2b, C3W, C3b):
  f32 = jnp.float32
  src = edge_index[0].astype(jnp.int32)
  dst = edge_index[1].astype(jnp.int32)

  pad_e = EP - E
  srcp = jnp.concatenate([src, jnp.zeros((pad_e,), jnp.int32)])
  srcx = jnp.concatenate([srcp, srcp + NP]).reshape(2 * NCHUNK, 128)
  dst2 = jnp.concatenate([dst, jnp.full((pad_e,), N, jnp.int32)]).reshape(NCHUNK, 128)

  xp = jnp.pad(x, ((0, NP - N), (0, 0)))
  x2 = jnp.concatenate([xp[:, :HALF], xp[:, HALF:]], axis=0)

  zrow = jnp.zeros((128, HALF), f32)
  onesr = jnp.ones((128, HALF), f32)

  deg = _sc_deg(dst2, zrow, onesr)
  agg1 = _sc_agg(x2, srcx, dst2, zrow).reshape(2, NP, HALF)

  wl1t = Wl1.T
  bl1b = jnp.broadcast_to(bl1[None, :], (8, D))
  h1 = _t1(agg1, deg, xp, wl1t[:HALF], wl1t[HALF:], Wr1.T, bl1b)

  agg2 = _sc_agg(h1.reshape(2 * NP, HALF), srcx, dst2, zrow).reshape(2, NP, HALF)

  wl2t = Wl2.T
  wr2t = Wr2.T
  bl2b = jnp.broadcast_to(bl2[None, :], (8, D))
  batch3 = jnp.concatenate([batch.astype(jnp.int32),
                            jnp.full((NP - N,), G, jnp.int32)]).reshape(NB, 1, B)
  c1wt = C1W.T
  c1bb = jnp.broadcast_to(C1b[None, :], (8, HALF))
  c2wt = jnp.zeros((HALF, HALF), f32).at[:, :64].set(C2W.T)
  c2bb = jnp.broadcast_to(jnp.zeros((HALF,), f32).at[:64].set(C2b)[None, :], (8, HALF))
  c3wt = jnp.zeros((HALF, HALF), f32).at[:64, 0].set(C3W[0])
  c3bb = jnp.broadcast_to(jnp.zeros((HALF,), f32).at[0].set(C3b[0])[None, :], (8, HALF))

  out128 = _t2(agg2, deg, h1, wl2t[:HALF], wl2t[HALF:],
               wr2t[:HALF], wr2t[HALF:], bl2b, batch3,
               c1wt, c1bb, c2wt, c2bb, c3wt, c3bb)
  return out128[:, :1]


# speedup vs baseline: 3.0773x; 1.0387x over previous
import functools

import jax
import jax.numpy as jnp
from jax import lax
from jax.experimental import pallas as pl
from jax.experimental.pallas import tpu as pltpu
from jax.experimental.pallas import tpu_sc as plsc

N = 10000
D = 256
E = 160000
G = 64
HALF = 128

NP = 10240
EP = 163840
NCHUNK = EP // 128
ROWS_PER_TILE = NP // 16
CHUNKS_PER_TILE = NCHUNK // 16
SUPERS = CHUNKS_PER_TILE // 8
NSTRIPE = ROWS_PER_TILE // 128

B = 512
NB = NP // B

_MESH = plsc.VectorSubcoreMesh(
    core_axis_name="c", subcore_axis_name="s", num_cores=2, num_subcores=16)


CH = 64
NBUF = 4
C64 = EP // CH
C64_PER_TILE = C64 // 16
SUP64 = 16
NSUP64 = C64_PER_TILE // SUP64


def _sc_agg_body(h2, srcx, dst2, zrow, agg_out, sidx, didx, r0, r1, r2, r3,
                 agg_sh, g0, g1, g2, g3, s0, s1, s2, s3):
  c = lax.axis_index("c")
  s = lax.axis_index("s")
  rbase = s * ROWS_PER_TILE

  pltpu.sync_copy(zrow.at[pl.ds(0, CH)], r0)
  pltpu.sync_copy(zrow.at[pl.ds(0, CH)], r1)
  for m in range(NSTRIPE):
    pltpu.sync_copy(r0, agg_sh.at[pl.ds(rbase + m * 128, 64)])
    pltpu.sync_copy(r1, agg_sh.at[pl.ds(rbase + m * 128 + 64, 64)])
  plsc.subcore_barrier()

  cb0 = s * C64_PER_TILE
  sbase = c * C64 + cb0

  rows = (r0, r1, r2, r3)
  gsem = (g0, g1, g2, g3)
  ssem = (s0, s1, s2, s3)

  def super_body(k):
    pltpu.sync_copy(srcx.at[pl.ds(sbase + k * SUP64, SUP64)], sidx)
    pltpu.sync_copy(dst2.at[pl.ds(cb0 + k * SUP64, SUP64)], didx)
    for u in range(NBUF):
      pltpu.async_copy(h2.at[sidx.at[u]], rows[u], gsem[u])
    for p in range(SUP64 - NBUF):
      u = p % NBUF
      pltpu.make_async_copy(h2.at[sidx.at[p]], rows[u], gsem[u]).wait()
      pltpu.async_copy(rows[u], agg_sh.at[didx.at[p]], ssem[u], add=True)
      pltpu.make_async_copy(rows[u], agg_sh.at[didx.at[p]], ssem[u]).wait()
      pltpu.async_copy(h2.at[sidx.at[p + NBUF]], rows[u], gsem[u])
    for p in range(SUP64 - NBUF, SUP64):
      u = p % NBUF
      pltpu.make_async_copy(h2.at[sidx.at[p]], rows[u], gsem[u]).wait()
      pltpu.async_copy(rows[u], agg_sh.at[didx.at[p]], ssem[u], add=True)
    for p in range(SUP64 - NBUF, SUP64):
      u = p % NBUF
      pltpu.make_async_copy(rows[u], agg_sh.at[didx.at[p]], ssem[u]).wait()

  pl.loop(0, NSUP64)(super_body)

  plsc.subcore_barrier()
  for m in range(NSTRIPE):
    pltpu.sync_copy(agg_sh.at[pl.ds(rbase + m * 128, 64)], r0)
    pltpu.sync_copy(r0, agg_out.at[pl.ds(c * NP + rbase + m * 128, 64)])
    pltpu.sync_copy(agg_sh.at[pl.ds(rbase + m * 128 + 64, 64)], r1)
    pltpu.sync_copy(r1, agg_out.at[pl.ds(c * NP + rbase + m * 128 + 64, 64)])


_sc_agg = pl.kernel(
    _sc_agg_body,
    out_type=jax.ShapeDtypeStruct((2 * NP, HALF), jnp.float32),
    mesh=_MESH,
    scratch_types=[
        pltpu.VMEM((SUP64, CH), jnp.int32),
        pltpu.VMEM((SUP64, CH), jnp.int32),
        pltpu.VMEM((CH, HALF), jnp.float32),
        pltpu.VMEM((CH, HALF), jnp.float32),
        pltpu.VMEM((CH, HALF), jnp.float32),
        pltpu.VMEM((CH, HALF), jnp.float32),
        pltpu.VMEM_SHARED((NP, HALF), jnp.float32),
        pltpu.SemaphoreType.DMA,
        pltpu.SemaphoreType.DMA,
        pltpu.SemaphoreType.DMA,
        pltpu.SemaphoreType.DMA,
        pltpu.SemaphoreType.DMA,
        pltpu.SemaphoreType.DMA,
        pltpu.SemaphoreType.DMA,
        pltpu.SemaphoreType.DMA,
    ],
)


def _sc_deg_body(dst2, zrow, onesr, deg_out, didx, onesv, buf, deg_sh):
  c = lax.axis_index("c")
  s = lax.axis_index("s")
  rbase = s * ROWS_PER_TILE

  @pl.when(c == 0)
  def _():
    pltpu.sync_copy(onesr.at[pl.ds(0, CH)], onesv)
    pltpu.sync_copy(zrow, buf)
    for m in range(NSTRIPE):
      pltpu.sync_copy(buf, deg_sh.at[pl.ds(rbase + m * 128, 128)])
  plsc.subcore_barrier()

  cb0 = s * C64_PER_TILE

  @pl.when(c == 0)
  def _():
    def super_body(k):
      pltpu.sync_copy(dst2.at[pl.ds(cb0 + k * SUP64, SUP64)], didx)
      for j in range(SUP64):
        pltpu.sync_copy(onesv, deg_sh.at[didx.at[j]], add=True)
    pl.loop(0, NSUP64)(super_body)

  plsc.subcore_barrier()

  @pl.when(c == 0)
  def _():
    for m in range(NSTRIPE):
      pltpu.sync_copy(deg_sh.at[pl.ds(rbase + m * 128, 128)], buf)
      pltpu.sync_copy(buf, deg_out.at[pl.ds(rbase + m * 128, 128)])


_sc_deg = pl.kernel(
    _sc_deg_body,
    out_type=jax.ShapeDtypeStruct((NP, HALF), jnp.float32),
    mesh=_MESH,
    scratch_types=[
        pltpu.VMEM((SUP64, CH), jnp.int32),
        pltpu.VMEM((CH, HALF), jnp.float32),
        pltpu.VMEM((128, HALF), jnp.float32),
        pltpu.VMEM_SHARED((NP, HALF), jnp.float32),
    ],
)


def _t1_body(agg_ref, deg_ref, x_ref, wl0_ref, wl1_ref, wr_ref, b_ref,
             out_ref):
  r = 1.0 / jnp.maximum(deg_ref[...], 1.0)
  a0 = agg_ref[0] * r
  a1 = agg_ref[1] * r
  h = jnp.dot(a0, wl0_ref[...], preferred_element_type=jnp.float32)
  h = h + jnp.dot(a1, wl1_ref[...], preferred_element_type=jnp.float32)
  h = h + jnp.dot(x_ref[...], wr_ref[...], preferred_element_type=jnp.float32)
  h = h + b_ref[0:1]
  h = jnp.maximum(h, 0.0)
  out_ref[0] = h[:, :HALF]
  out_ref[1] = h[:, HALF:]


_t1 = pl.pallas_call(
    _t1_body,
    grid=(NB,),
    in_specs=[
        pl.BlockSpec((2, B, HALF), lambda i: (0, i, 0)),
        pl.BlockSpec((B, HALF), lambda i: (i, 0)),
        pl.BlockSpec((B, D), lambda i: (i, 0)),
        pl.BlockSpec((HALF, D), lambda i: (0, 0)),
        pl.BlockSpec((HALF, D), lambda i: (0, 0)),
        pl.BlockSpec((D, D), lambda i: (0, 0)),
        pl.BlockSpec((8, D), lambda i: (0, 0)),
    ],
    out_specs=pl.BlockSpec((2, B, HALF), lambda i: (0, i, 0)),
    out_shape=jax.ShapeDtypeStruct((2, NP, HALF), jnp.float32),
)


def _t2_body(agg_ref, deg_ref, h1_ref, wl0_ref, wl1_ref, wr0_ref, wr1_ref,
             b_ref, batch_ref, c1w_ref, c1b_ref, c2w_ref, c2b_ref, c3w_ref,
             c3b_ref, out_ref, psum, cnt):
  i = pl.program_id(0)

  @pl.when(i == 0)
  def _():
    psum[...] = jnp.zeros_like(psum)
    cnt[...] = jnp.zeros_like(cnt)

  r = 1.0 / jnp.maximum(deg_ref[...], 1.0)
  a0 = agg_ref[0] * r
  a1 = agg_ref[1] * r
  h = jnp.dot(a0, wl0_ref[...], preferred_element_type=jnp.float32)
  h = h + jnp.dot(a1, wl1_ref[...], preferred_element_type=jnp.float32)
  h = h + jnp.dot(h1_ref[0], wr0_ref[...], preferred_element_type=jnp.float32)
  h = h + jnp.dot(h1_ref[1], wr1_ref[...], preferred_element_type=jnp.float32)
  h = h + b_ref[0:1]
  h2 = jnp.maximum(h, 0.0)

  b = batch_ref[0]
  gid = lax.broadcasted_iota(jnp.int32, (G, B), 0)
  rowid = lax.broadcasted_iota(jnp.int32, (G, B), 1) + i * B
  mask = jnp.where((b == gid) & (rowid < N), 1.0, 0.0)
  psum[...] += jnp.dot(mask, h2, preferred_element_type=jnp.float32)
  cnt[...] += jnp.sum(mask, axis=1, keepdims=True)

  @pl.when(i == NB - 1)
  def _():
    cw = cnt[:, 0:1]
    pooled = psum[...] / jnp.maximum(cw, 1.0)
    z = jnp.dot(pooled, c1w_ref[...], preferred_element_type=jnp.float32)
    z = jnp.maximum(z + c1b_ref[0:1], 0.0)
    z = jnp.dot(z, c2w_ref[...], preferred_element_type=jnp.float32)
    z = jnp.maximum(z + c2b_ref[0:1], 0.0)
    o = jnp.dot(z, c3w_ref[...], preferred_element_type=jnp.float32)
    out_ref[...] = o + c3b_ref[0:1]


_t2 = pl.pallas_call(
    _t2_body,
    grid=(NB,),
    in_specs=[
        pl.BlockSpec((2, B, HALF), lambda i: (0, i, 0)),
        pl.BlockSpec((B, HALF), lambda i: (i, 0)),
        pl.BlockSpec((2, B, HALF), lambda i: (0, i, 0)),
        pl.BlockSpec((HALF, D), lambda i: (0, 0)),
        pl.BlockSpec((HALF, D), lambda i: (0, 0)),
        pl.BlockSpec((HALF, D), lambda i: (0, 0)),
        pl.BlockSpec((HALF, D), lambda i: (0, 0)),
        pl.BlockSpec((8, D), lambda i: (0, 0)),
        pl.BlockSpec((1, 1, B), lambda i: (i, 0, 0)),
        pl.BlockSpec((D, HALF), lambda i: (0, 0)),
        pl.BlockSpec((8, HALF), lambda i: (0, 0)),
        pl.BlockSpec((HALF, HALF), lambda i: (0, 0)),
        pl.BlockSpec((8, HALF), lambda i: (0, 0)),
        pl.BlockSpec((HALF, HALF), lambda i: (0, 0)),
        pl.BlockSpec((8, HALF), lambda i: (0, 0)),
    ],
    out_specs=pl.BlockSpec((G, HALF), lambda i: (0, 0)),
    out_shape=jax.ShapeDtypeStruct((G, HALF), jnp.float32),
    scratch_shapes=[
        pltpu.VMEM((G, D), jnp.float32),
        pltpu.VMEM((G, HALF), jnp.float32),
    ],
)


def kernel(x, edge_index, batch, Wl1, bl1, Wr1, Wl2, bl2, Wr2,
           C1W, C1b, C2W, C2b, C3W, C3b):
  f32 = jnp.float32
  src = edge_index[0].astype(jnp.int32)
  dst = edge_index[1].astype(jnp.int32)

  pad_e = EP - E
  srcp = jnp.concatenate([src, jnp.zeros((pad_e,), jnp.int32)])
  srcx = jnp.concatenate([srcp, srcp + NP]).reshape(2 * C64, CH)
  dst2 = jnp.concatenate([dst, jnp.full((pad_e,), N, jnp.int32)]).reshape(C64, CH)

  xp = jnp.pad(x, ((0, NP - N), (0, 0)))
  x2 = jnp.concatenate([xp[:, :HALF], xp[:, HALF:]], axis=0)

  zrow = jnp.zeros((128, HALF), f32)
  onesr = jnp.ones((128, HALF), f32)

  deg = _sc_deg(dst2, zrow, onesr)
  agg1 = _sc_agg(x2, srcx, dst2, zrow).reshape(2, NP, HALF)

  wl1t = Wl1.T
  bl1b = jnp.broadcast_to(bl1[None, :], (8, D))
  h1 = _t1(agg1, deg, xp, wl1t[:HALF], wl1t[HALF:], Wr1.T, bl1b)

  agg2 = _sc_agg(h1.reshape(2 * NP, HALF), srcx, dst2, zrow).reshape(2, NP, HALF)

  wl2t = Wl2.T
  wr2t = Wr2.T
  bl2b = jnp.broadcast_to(bl2[None, :], (8, D))
  batch3 = jnp.concatenate([batch.astype(jnp.int32),
                            jnp.full((NP - N,), G, jnp.int32)]).reshape(NB, 1, B)
  c1wt = C1W.T
  c1bb = jnp.broadcast_to(C1b[None, :], (8, HALF))
  c2wt = jnp.zeros((HALF, HALF), f32).at[:, :64].set(C2W.T)
  c2bb = jnp.broadcast_to(jnp.zeros((HALF,), f32).at[:64].set(C2b)[None, :], (8, HALF))
  c3wt = jnp.zeros((HALF, HALF), f32).at[:64, 0].set(C3W[0])
  c3bb = jnp.broadcast_to(jnp.zeros((HALF,), f32).at[0].set(C3b[0])[None, :], (8, HALF))

  out128 = _t2(agg2, deg, h1, wl2t[:HALF], wl2t[HALF:],
               wr2t[:HALF], wr2t[HALF:], bl2b, batch3,
               c1wt, c1bb, c2wt, c2bb, c3wt, c3bb)
  return out128[:, :1]


# speedup vs baseline: 3.1889x; 1.0362x over previous
import functools

import jax
import jax.numpy as jnp
from jax import lax
from jax.experimental import pallas as pl
from jax.experimental.pallas import tpu as pltpu
from jax.experimental.pallas import tpu_sc as plsc

N = 10000
D = 256
E = 160000
G = 64
HALF = 128

NP = 10240
EP = 163840
NCHUNK = EP // 128
ROWS_PER_TILE = NP // 16
CHUNKS_PER_TILE = NCHUNK // 16
SUPERS = CHUNKS_PER_TILE // 8
NSTRIPE = ROWS_PER_TILE // 128

B = 512
NB = NP // B

_MESH = plsc.VectorSubcoreMesh(
    core_axis_name="c", subcore_axis_name="s", num_cores=2, num_subcores=16)


CH = 64
NBUF = 4
C64 = EP // CH
C64_PER_TILE = C64 // 16
SUP64 = 16
NSUP64 = C64_PER_TILE // SUP64


def _sc_agg_body(h2, srcx, dst2, zrow, agg_out, sidx, didx, r0, r1, r2, r3,
                 agg_sh, g0, g1, g2, g3, s0, s1, s2, s3):
  c = lax.axis_index("c")
  s = lax.axis_index("s")
  rbase = s * ROWS_PER_TILE

  pltpu.sync_copy(zrow.at[pl.ds(0, CH)], r0)
  pltpu.sync_copy(zrow.at[pl.ds(0, CH)], r1)
  for m in range(NSTRIPE):
    pltpu.sync_copy(r0, agg_sh.at[pl.ds(rbase + m * 128, 64)])
    pltpu.sync_copy(r1, agg_sh.at[pl.ds(rbase + m * 128 + 64, 64)])
  plsc.subcore_barrier()

  cb0 = s * C64_PER_TILE
  sbase = c * C64 + cb0

  rows = (r0, r1, r2, r3)
  gsem = (g0, g1, g2, g3)
  ssem = (s0, s1, s2, s3)

  def super_body(k):
    pltpu.sync_copy(srcx.at[pl.ds(sbase + k * SUP64, SUP64)], sidx)
    pltpu.sync_copy(dst2.at[pl.ds(cb0 + k * SUP64, SUP64)], didx)
    for u in range(NBUF):
      pltpu.async_copy(h2.at[sidx.at[u]], rows[u], gsem[u])
    for p in range(SUP64 - NBUF):
      u = p % NBUF
      pltpu.make_async_copy(h2.at[sidx.at[p]], rows[u], gsem[u]).wait()
      pltpu.sync_copy(rows[u], agg_sh.at[didx.at[p]], add=True)
      pltpu.async_copy(h2.at[sidx.at[p + NBUF]], rows[u], gsem[u])
    for p in range(SUP64 - NBUF, SUP64):
      u = p % NBUF
      pltpu.make_async_copy(h2.at[sidx.at[p]], rows[u], gsem[u]).wait()
      pltpu.sync_copy(rows[u], agg_sh.at[didx.at[p]], add=True)

  pl.loop(0, NSUP64)(super_body)

  plsc.subcore_barrier()
  for m in range(NSTRIPE):
    pltpu.sync_copy(agg_sh.at[pl.ds(rbase + m * 128, 64)], r0)
    pltpu.sync_copy(r0, agg_out.at[pl.ds(c * NP + rbase + m * 128, 64)])
    pltpu.sync_copy(agg_sh.at[pl.ds(rbase + m * 128 + 64, 64)], r1)
    pltpu.sync_copy(r1, agg_out.at[pl.ds(c * NP + rbase + m * 128 + 64, 64)])


_sc_agg = pl.kernel(
    _sc_agg_body,
    out_type=jax.ShapeDtypeStruct((2 * NP, HALF), jnp.float32),
    mesh=_MESH,
    scratch_types=[
        pltpu.VMEM((SUP64, CH), jnp.int32),
        pltpu.VMEM((SUP64, CH), jnp.int32),
        pltpu.VMEM((CH, HALF), jnp.float32),
        pltpu.VMEM((CH, HALF), jnp.float32),
        pltpu.VMEM((CH, HALF), jnp.float32),
        pltpu.VMEM((CH, HALF), jnp.float32),
        pltpu.VMEM_SHARED((NP, HALF), jnp.float32),
        pltpu.SemaphoreType.DMA,
        pltpu.SemaphoreType.DMA,
        pltpu.SemaphoreType.DMA,
        pltpu.SemaphoreType.DMA,
        pltpu.SemaphoreType.DMA,
        pltpu.SemaphoreType.DMA,
        pltpu.SemaphoreType.DMA,
        pltpu.SemaphoreType.DMA,
    ],
)


def _sc_deg_body(dst2, zrow, onesr, deg_out, didx, onesv, buf, deg_sh):
  c = lax.axis_index("c")
  s = lax.axis_index("s")
  rbase = s * ROWS_PER_TILE

  pltpu.sync_copy(onesr.at[pl.ds(0, CH)], onesv)
  pltpu.sync_copy(zrow, buf)
  for m in range(NSTRIPE):
    pltpu.sync_copy(buf, deg_sh.at[pl.ds(rbase + m * 128, 128)])
  plsc.subcore_barrier()

  chalf = C64 // 2
  cb0 = c * chalf + s * (chalf // 16)

  def super_body(k):
    pltpu.sync_copy(dst2.at[pl.ds(cb0 + k * SUP64, SUP64)], didx)
    for j in range(SUP64):
      pltpu.sync_copy(onesv, deg_sh.at[didx.at[j]], add=True)

  pl.loop(0, chalf // 16 // SUP64)(super_body)

  plsc.subcore_barrier()
  for m in range(NSTRIPE):
    pltpu.sync_copy(deg_sh.at[pl.ds(rbase + m * 128, 128)], buf)
    pltpu.sync_copy(buf, deg_out.at[pl.ds(c * NP + rbase + m * 128, 128)])


_sc_deg = pl.kernel(
    _sc_deg_body,
    out_type=jax.ShapeDtypeStruct((2 * NP, HALF), jnp.float32),
    mesh=_MESH,
    scratch_types=[
        pltpu.VMEM((SUP64, CH), jnp.int32),
        pltpu.VMEM((CH, HALF), jnp.float32),
        pltpu.VMEM((128, HALF), jnp.float32),
        pltpu.VMEM_SHARED((NP, HALF), jnp.float32),
    ],
)


def _t1_body(agg_ref, deg_ref, x_ref, wl0_ref, wl1_ref, wr_ref, b_ref,
             out_ref):
  r = 1.0 / jnp.maximum(deg_ref[0] + deg_ref[1], 1.0)
  a0 = agg_ref[0] * r
  a1 = agg_ref[1] * r
  h = jnp.dot(a0, wl0_ref[...], preferred_element_type=jnp.float32)
  h = h + jnp.dot(a1, wl1_ref[...], preferred_element_type=jnp.float32)
  h = h + jnp.dot(x_ref[...], wr_ref[...], preferred_element_type=jnp.float32)
  h = h + b_ref[0:1]
  h = jnp.maximum(h, 0.0)
  out_ref[0] = h[:, :HALF]
  out_ref[1] = h[:, HALF:]


_t1 = pl.pallas_call(
    _t1_body,
    grid=(NB,),
    in_specs=[
        pl.BlockSpec((2, B, HALF), lambda i: (0, i, 0)),
        pl.BlockSpec((2, B, HALF), lambda i: (0, i, 0)),
        pl.BlockSpec((B, D), lambda i: (i, 0)),
        pl.BlockSpec((HALF, D), lambda i: (0, 0)),
        pl.BlockSpec((HALF, D), lambda i: (0, 0)),
        pl.BlockSpec((D, D), lambda i: (0, 0)),
        pl.BlockSpec((8, D), lambda i: (0, 0)),
    ],
    out_specs=pl.BlockSpec((2, B, HALF), lambda i: (0, i, 0)),
    out_shape=jax.ShapeDtypeStruct((2, NP, HALF), jnp.float32),
)


def _t2_body(agg_ref, deg_ref, h1_ref, wl0_ref, wl1_ref, wr0_ref, wr1_ref,
             b_ref, batch_ref, c1w_ref, c1b_ref, c2w_ref, c2b_ref, c3w_ref,
             c3b_ref, out_ref, psum, cnt):
  i = pl.program_id(0)

  @pl.when(i == 0)
  def _():
    psum[...] = jnp.zeros_like(psum)
    cnt[...] = jnp.zeros_like(cnt)

  r = 1.0 / jnp.maximum(deg_ref[0] + deg_ref[1], 1.0)
  a0 = agg_ref[0] * r
  a1 = agg_ref[1] * r
  h = jnp.dot(a0, wl0_ref[...], preferred_element_type=jnp.float32)
  h = h + jnp.dot(a1, wl1_ref[...], preferred_element_type=jnp.float32)
  h = h + jnp.dot(h1_ref[0], wr0_ref[...], preferred_element_type=jnp.float32)
  h = h + jnp.dot(h1_ref[1], wr1_ref[...], preferred_element_type=jnp.float32)
  h = h + b_ref[0:1]
  h2 = jnp.maximum(h, 0.0)

  b = batch_ref[0]
  gid = lax.broadcasted_iota(jnp.int32, (G, B), 0)
  rowid = lax.broadcasted_iota(jnp.int32, (G, B), 1) + i * B
  mask = jnp.where((b == gid) & (rowid < N), 1.0, 0.0)
  psum[...] += jnp.dot(mask, h2, preferred_element_type=jnp.float32)
  cnt[...] += jnp.sum(mask, axis=1, keepdims=True)

  @pl.when(i == NB - 1)
  def _():
    cw = cnt[:, 0:1]
    pooled = psum[...] / jnp.maximum(cw, 1.0)
    z = jnp.dot(pooled, c1w_ref[...], preferred_element_type=jnp.float32)
    z = jnp.maximum(z + c1b_ref[0:1], 0.0)
    z = jnp.dot(z, c2w_ref[...], preferred_element_type=jnp.float32)
    z = jnp.maximum(z + c2b_ref[0:1], 0.0)
    o = jnp.dot(z, c3w_ref[...], preferred_element_type=jnp.float32)
    out_ref[...] = o + c3b_ref[0:1]


_t2 = pl.pallas_call(
    _t2_body,
    grid=(NB,),
    in_specs=[
        pl.BlockSpec((2, B, HALF), lambda i: (0, i, 0)),
        pl.BlockSpec((2, B, HALF), lambda i: (0, i, 0)),
        pl.BlockSpec((2, B, HALF), lambda i: (0, i, 0)),
        pl.BlockSpec((HALF, D), lambda i: (0, 0)),
        pl.BlockSpec((HALF, D), lambda i: (0, 0)),
        pl.BlockSpec((HALF, D), lambda i: (0, 0)),
        pl.BlockSpec((HALF, D), lambda i: (0, 0)),
        pl.BlockSpec((8, D), lambda i: (0, 0)),
        pl.BlockSpec((1, 1, B), lambda i: (i, 0, 0)),
        pl.BlockSpec((D, HALF), lambda i: (0, 0)),
        pl.BlockSpec((8, HALF), lambda i: (0, 0)),
        pl.BlockSpec((HALF, HALF), lambda i: (0, 0)),
        pl.BlockSpec((8, HALF), lambda i: (0, 0)),
        pl.BlockSpec((HALF, HALF), lambda i: (0, 0)),
        pl.BlockSpec((8, HALF), lambda i: (0, 0)),
    ],
    out_specs=pl.BlockSpec((G, HALF), lambda i: (0, 0)),
    out_shape=jax.ShapeDtypeStruct((G, HALF), jnp.float32),
    scratch_shapes=[
        pltpu.VMEM((G, D), jnp.float32),
        pltpu.VMEM((G, HALF), jnp.float32),
    ],
)


def kernel(x, edge_index, batch, Wl1, bl1, Wr1, Wl2, bl2, Wr2,
           C1W, C1b, C2W, C2b, C3W, C3b):
  f32 = jnp.float32
  src = edge_index[0].astype(jnp.int32)
  dst = edge_index[1].astype(jnp.int32)

  pad_e = EP - E
  srcp = jnp.concatenate([src, jnp.zeros((pad_e,), jnp.int32)])
  srcx = jnp.concatenate([srcp, srcp + NP]).reshape(2 * C64, CH)
  dst2 = jnp.concatenate([dst, jnp.full((pad_e,), N, jnp.int32)]).reshape(C64, CH)

  xp = jnp.pad(x, ((0, NP - N), (0, 0)))
  x2 = jnp.concatenate([xp[:, :HALF], xp[:, HALF:]], axis=0)

  zrow = jnp.zeros((128, HALF), f32)
  onesr = jnp.ones((128, HALF), f32)

  deg = _sc_deg(dst2, zrow, onesr).reshape(2, NP, HALF)
  agg1 = _sc_agg(x2, srcx, dst2, zrow).reshape(2, NP, HALF)

  wl1t = Wl1.T
  bl1b = jnp.broadcast_to(bl1[None, :], (8, D))
  h1 = _t1(agg1, deg, xp, wl1t[:HALF], wl1t[HALF:], Wr1.T, bl1b)

  agg2 = _sc_agg(h1.reshape(2 * NP, HALF), srcx, dst2, zrow).reshape(2, NP, HALF)

  wl2t = Wl2.T
  wr2t = Wr2.T
  bl2b = jnp.broadcast_to(bl2[None, :], (8, D))
  batch3 = jnp.concatenate([batch.astype(jnp.int32),
                            jnp.full((NP - N,), G, jnp.int32)]).reshape(NB, 1, B)
  c1wt = C1W.T
  c1bb = jnp.broadcast_to(C1b[None, :], (8, HALF))
  c2wt = jnp.zeros((HALF, HALF), f32).at[:, :64].set(C2W.T)
  c2bb = jnp.broadcast_to(jnp.zeros((HALF,), f32).at[:64].set(C2b)[None, :], (8, HALF))
  c3wt = jnp.zeros((HALF, HALF), f32).at[:64, 0].set(C3W[0])
  c3bb = jnp.broadcast_to(jnp.zeros((HALF,), f32).at[0].set(C3b[0])[None, :], (8, HALF))

  out128 = _t2(agg2, deg, h1, wl2t[:HALF], wl2t[HALF:],
               wr2t[:HALF], wr2t[HALF:], bl2b, batch3,
               c1wt, c1bb, c2wt, c2bb, c3wt, c3bb)
  return out128[:, :1]


# speedup vs baseline: 3.2333x; 1.0139x over previous
import functools

import jax
import jax.numpy as jnp
from jax import lax
from jax.experimental import pallas as pl
from jax.experimental.pallas import tpu as pltpu
from jax.experimental.pallas import tpu_sc as plsc

N = 10000
D = 256
E = 160000
G = 64
HALF = 128

NP = 10240
EP = 163840
NCHUNK = EP // 128
ROWS_PER_TILE = NP // 16
CHUNKS_PER_TILE = NCHUNK // 16
SUPERS = CHUNKS_PER_TILE // 8
NSTRIPE = ROWS_PER_TILE // 128

B = 512
NB = NP // B

_MESH = plsc.VectorSubcoreMesh(
    core_axis_name="c", subcore_axis_name="s", num_cores=2, num_subcores=16)


CH = 128
NBUF = 2
C64 = EP // CH
C64_PER_TILE = C64 // 16
SUP64 = 8
NSUP64 = C64_PER_TILE // SUP64


def _sc_agg_body(h2, srcx, dst2, zrow, agg_out, sidx, didx, r0, r1,
                 agg_sh, g0, g1):
  c = lax.axis_index("c")
  s = lax.axis_index("s")
  rbase = s * ROWS_PER_TILE

  pltpu.sync_copy(zrow, r0)
  for m in range(NSTRIPE):
    pltpu.sync_copy(r0, agg_sh.at[pl.ds(rbase + m * 128, 128)])
  plsc.subcore_barrier()

  cb0 = s * C64_PER_TILE
  sbase = c * C64 + cb0

  rows = (r0, r1)
  gsem = (g0, g1)

  def super_body(k):
    pltpu.sync_copy(srcx.at[pl.ds(sbase + k * SUP64, SUP64)], sidx)
    pltpu.sync_copy(dst2.at[pl.ds(cb0 + k * SUP64, SUP64)], didx)
    for u in range(NBUF):
      pltpu.async_copy(h2.at[sidx.at[u]], rows[u], gsem[u])
    for p in range(SUP64 - NBUF):
      u = p % NBUF
      pltpu.make_async_copy(h2.at[sidx.at[p]], rows[u], gsem[u]).wait()
      pltpu.sync_copy(rows[u], agg_sh.at[didx.at[p]], add=True)
      pltpu.async_copy(h2.at[sidx.at[p + NBUF]], rows[u], gsem[u])
    for p in range(SUP64 - NBUF, SUP64):
      u = p % NBUF
      pltpu.make_async_copy(h2.at[sidx.at[p]], rows[u], gsem[u]).wait()
      pltpu.sync_copy(rows[u], agg_sh.at[didx.at[p]], add=True)

  pl.loop(0, NSUP64)(super_body)

  plsc.subcore_barrier()
  for m in range(NSTRIPE):
    pltpu.sync_copy(agg_sh.at[pl.ds(rbase + m * 128, 128)], r0)
    pltpu.sync_copy(r0, agg_out.at[pl.ds(c * NP + rbase + m * 128, 128)])


_sc_agg = pl.kernel(
    _sc_agg_body,
    out_type=jax.ShapeDtypeStruct((2 * NP, HALF), jnp.float32),
    mesh=_MESH,
    scratch_types=[
        pltpu.VMEM((SUP64, CH), jnp.int32),
        pltpu.VMEM((SUP64, CH), jnp.int32),
        pltpu.VMEM((CH, HALF), jnp.float32),
        pltpu.VMEM((CH, HALF), jnp.float32),
        pltpu.VMEM_SHARED((NP, HALF), jnp.float32),
        pltpu.SemaphoreType.DMA,
        pltpu.SemaphoreType.DMA,
    ],
)


def _sc_deg_body(dst2, zrow, onesr, deg_out, didx, onesv, buf, deg_sh):
  c = lax.axis_index("c")
  s = lax.axis_index("s")
  rbase = s * ROWS_PER_TILE

  pltpu.sync_copy(onesr.at[pl.ds(0, CH)], onesv)
  pltpu.sync_copy(zrow, buf)
  for m in range(NSTRIPE):
    pltpu.sync_copy(buf, deg_sh.at[pl.ds(rbase + m * 128, 128)])
  plsc.subcore_barrier()

  chalf = C64 // 2
  cb0 = c * chalf + s * (chalf // 16)

  def super_body(k):
    pltpu.sync_copy(dst2.at[pl.ds(cb0 + k * SUP64, SUP64)], didx)
    for j in range(SUP64):
      pltpu.sync_copy(onesv, deg_sh.at[didx.at[j]], add=True)

  pl.loop(0, chalf // 16 // SUP64)(super_body)

  plsc.subcore_barrier()
  for m in range(NSTRIPE):
    pltpu.sync_copy(deg_sh.at[pl.ds(rbase + m * 128, 128)], buf)
    pltpu.sync_copy(buf, deg_out.at[pl.ds(c * NP + rbase + m * 128, 128)])


_sc_deg = pl.kernel(
    _sc_deg_body,
    out_type=jax.ShapeDtypeStruct((2 * NP, HALF), jnp.float32),
    mesh=_MESH,
    scratch_types=[
        pltpu.VMEM((SUP64, CH), jnp.int32),
        pltpu.VMEM((CH, HALF), jnp.float32),
        pltpu.VMEM((128, HALF), jnp.float32),
        pltpu.VMEM_SHARED((NP, HALF), jnp.float32),
    ],
)


def _t1_body(agg_ref, deg_ref, x_ref, wl0_ref, wl1_ref, wr_ref, b_ref,
             out_ref):
  r = 1.0 / jnp.maximum(deg_ref[0] + deg_ref[1], 1.0)
  a0 = agg_ref[0] * r
  a1 = agg_ref[1] * r
  h = jnp.dot(a0, wl0_ref[...], preferred_element_type=jnp.float32)
  h = h + jnp.dot(a1, wl1_ref[...], preferred_element_type=jnp.float32)
  h = h + jnp.dot(x_ref[...], wr_ref[...], preferred_element_type=jnp.float32)
  h = h + b_ref[0:1]
  h = jnp.maximum(h, 0.0)
  out_ref[0] = h[:, :HALF]
  out_ref[1] = h[:, HALF:]


_t1 = pl.pallas_call(
    _t1_body,
    grid=(NB,),
    in_specs=[
        pl.BlockSpec((2, B, HALF), lambda i: (0, i, 0)),
        pl.BlockSpec((2, B, HALF), lambda i: (0, i, 0)),
        pl.BlockSpec((B, D), lambda i: (i, 0)),
        pl.BlockSpec((HALF, D), lambda i: (0, 0)),
        pl.BlockSpec((HALF, D), lambda i: (0, 0)),
        pl.BlockSpec((D, D), lambda i: (0, 0)),
        pl.BlockSpec((8, D), lambda i: (0, 0)),
    ],
    out_specs=pl.BlockSpec((2, B, HALF), lambda i: (0, i, 0)),
    out_shape=jax.ShapeDtypeStruct((2, NP, HALF), jnp.float32),
)


def _t2_body(agg_ref, deg_ref, h1_ref, wl0_ref, wl1_ref, wr0_ref, wr1_ref,
             b_ref, batch_ref, c1w_ref, c1b_ref, c2w_ref, c2b_ref, c3w_ref,
             c3b_ref, out_ref, psum, cnt):
  i = pl.program_id(0)

  @pl.when(i == 0)
  def _():
    psum[...] = jnp.zeros_like(psum)
    cnt[...] = jnp.zeros_like(cnt)

  r = 1.0 / jnp.maximum(deg_ref[0] + deg_ref[1], 1.0)
  a0 = agg_ref[0] * r
  a1 = agg_ref[1] * r
  h = jnp.dot(a0, wl0_ref[...], preferred_element_type=jnp.float32)
  h = h + jnp.dot(a1, wl1_ref[...], preferred_element_type=jnp.float32)
  h = h + jnp.dot(h1_ref[0], wr0_ref[...], preferred_element_type=jnp.float32)
  h = h + jnp.dot(h1_ref[1], wr1_ref[...], preferred_element_type=jnp.float32)
  h = h + b_ref[0:1]
  h2 = jnp.maximum(h, 0.0)

  b = batch_ref[0]
  gid = lax.broadcasted_iota(jnp.int32, (G, B), 0)
  rowid = lax.broadcasted_iota(jnp.int32, (G, B), 1) + i * B
  mask = jnp.where((b == gid) & (rowid < N), 1.0, 0.0)
  psum[...] += jnp.dot(mask, h2, preferred_element_type=jnp.float32)
  cnt[...] += jnp.sum(mask, axis=1, keepdims=True)

  @pl.when(i == NB - 1)
  def _():
    cw = cnt[:, 0:1]
    pooled = psum[...] / jnp.maximum(cw, 1.0)
    z = jnp.dot(pooled, c1w_ref[...], preferred_element_type=jnp.float32)
    z = jnp.maximum(z + c1b_ref[0:1], 0.0)
    z = jnp.dot(z, c2w_ref[...], preferred_element_type=jnp.float32)
    z = jnp.maximum(z + c2b_ref[0:1], 0.0)
    o = jnp.dot(z, c3w_ref[...], preferred_element_type=jnp.float32)
    out_ref[...] = o + c3b_ref[0:1]


_t2 = pl.pallas_call(
    _t2_body,
    grid=(NB,),
    in_specs=[
        pl.BlockSpec((2, B, HALF), lambda i: (0, i, 0)),
        pl.BlockSpec((2, B, HALF), lambda i: (0, i, 0)),
        pl.BlockSpec((2, B, HALF), lambda i: (0, i, 0)),
        pl.BlockSpec((HALF, D), lambda i: (0, 0)),
        pl.BlockSpec((HALF, D), lambda i: (0, 0)),
        pl.BlockSpec((HALF, D), lambda i: (0, 0)),
        pl.BlockSpec((HALF, D), lambda i: (0, 0)),
        pl.BlockSpec((8, D), lambda i: (0, 0)),
        pl.BlockSpec((1, 1, B), lambda i: (i, 0, 0)),
        pl.BlockSpec((D, HALF), lambda i: (0, 0)),
        pl.BlockSpec((8, HALF), lambda i: (0, 0)),
        pl.BlockSpec((HALF, HALF), lambda i: (0, 0)),
        pl.BlockSpec((8, HALF), lambda i: (0, 0)),
        pl.BlockSpec((HALF, HALF), lambda i: (0, 0)),
        pl.BlockSpec((8, HALF), lambda i: (0, 0)),
    ],
    out_specs=pl.BlockSpec((G, HALF), lambda i: (0, 0)),
    out_shape=jax.ShapeDtypeStruct((G, HALF), jnp.float32),
    scratch_shapes=[
        pltpu.VMEM((G, D), jnp.float32),
        pltpu.VMEM((G, HALF), jnp.float32),
    ],
)


def kernel(x, edge_index, batch, Wl1, bl1, Wr1, Wl2, bl2, Wr2,
           C1W, C1b, C2W, C2b, C3W, C3b):
  f32 = jnp.float32
  src = edge_index[0].astype(jnp.int32)
  dst = edge_index[1].astype(jnp.int32)

  pad_e = EP - E
  srcp = jnp.concatenate([src, jnp.zeros((pad_e,), jnp.int32)])
  srcx = jnp.concatenate([srcp, srcp + NP]).reshape(2 * C64, CH)
  dst2 = jnp.concatenate([dst, jnp.full((pad_e,), N, jnp.int32)]).reshape(C64, CH)

  xp = jnp.pad(x, ((0, NP - N), (0, 0)))
  x2 = jnp.concatenate([xp[:, :HALF], xp[:, HALF:]], axis=0)

  zrow = jnp.zeros((128, HALF), f32)
  onesr = jnp.ones((128, HALF), f32)

  deg = _sc_deg(dst2, zrow, onesr).reshape(2, NP, HALF)
  agg1 = _sc_agg(x2, srcx, dst2, zrow).reshape(2, NP, HALF)

  wl1t = Wl1.T
  bl1b = jnp.broadcast_to(bl1[None, :], (8, D))
  h1 = _t1(agg1, deg, xp, wl1t[:HALF], wl1t[HALF:], Wr1.T, bl1b)

  agg2 = _sc_agg(h1.reshape(2 * NP, HALF), srcx, dst2, zrow).reshape(2, NP, HALF)

  wl2t = Wl2.T
  wr2t = Wr2.T
  bl2b = jnp.broadcast_to(bl2[None, :], (8, D))
  batch3 = jnp.concatenate([batch.astype(jnp.int32),
                            jnp.full((NP - N,), G, jnp.int32)]).reshape(NB, 1, B)
  c1wt = C1W.T
  c1bb = jnp.broadcast_to(C1b[None, :], (8, HALF))
  c2wt = jnp.zeros((HALF, HALF), f32).at[:, :64].set(C2W.T)
  c2bb = jnp.broadcast_to(jnp.zeros((HALF,), f32).at[:64].set(C2b)[None, :], (8, HALF))
  c3wt = jnp.zeros((HALF, HALF), f32).at[:64, 0].set(C3W[0])
  c3bb = jnp.broadcast_to(jnp.zeros((HALF,), f32).at[0].set(C3b[0])[None, :], (8, HALF))

  out128 = _t2(agg2, deg, h1, wl2t[:HALF], wl2t[HALF:],
               wr2t[:HALF], wr2t[HALF:], bl2b, batch3,
               c1wt, c1bb, c2wt, c2bb, c3wt, c3bb)
  return out128[:, :1]


# speedup vs baseline: 3.4018x; 1.0521x over previous
import functools

import jax
import jax.numpy as jnp
from jax import lax
from jax.experimental import pallas as pl
from jax.experimental.pallas import tpu as pltpu
from jax.experimental.pallas import tpu_sc as plsc

N = 10000
D = 256
E = 160000
G = 64
HALF = 128

NP = 10240
EP = 163840
NCHUNK = EP // 128
ROWS_PER_TILE = NP // 16
CHUNKS_PER_TILE = NCHUNK // 16
SUPERS = CHUNKS_PER_TILE // 8
NSTRIPE = ROWS_PER_TILE // 128

B = 512
NB = NP // B

_MESH = plsc.VectorSubcoreMesh(
    core_axis_name="c", subcore_axis_name="s", num_cores=2, num_subcores=16)


CH = 128
NBUF = 2
C64 = EP // CH
C64_PER_TILE = C64 // 16
SUP64 = 8
NSUP64 = C64_PER_TILE // SUP64


def _sc_agg_body(h2, srcx, dst2, zrow, agg_out, sidx, didx, r0, r1,
                 agg_sh, g0, g1):
  c = lax.axis_index("c")
  s = lax.axis_index("s")
  rbase = s * ROWS_PER_TILE

  pltpu.sync_copy(zrow, r0)
  for m in range(NSTRIPE):
    pltpu.sync_copy(r0, agg_sh.at[pl.ds(rbase + m * 128, 128)])
  plsc.subcore_barrier()

  cb0 = s * C64_PER_TILE
  sbase = c * C64 + cb0

  rows = (r0, r1)
  gsem = (g0, g1)

  pltpu.sync_copy(srcx.at[pl.ds(sbase, C64_PER_TILE)], sidx)
  for u in range(NBUF):
    pltpu.async_copy(h2.at[sidx.at[u]], rows[u], gsem[u])

  def super_body(k):
    kb = k * SUP64
    pltpu.sync_copy(dst2.at[pl.ds(cb0 + kb, SUP64)], didx)
    for p in range(SUP64):
      u = p % NBUF
      pltpu.make_async_copy(h2.at[sidx.at[kb + p]], rows[u], gsem[u]).wait()
      pltpu.sync_copy(rows[u], agg_sh.at[didx.at[p]], add=True)

      @pl.when(kb + p + NBUF < C64_PER_TILE)
      def _():
        pltpu.async_copy(h2.at[sidx.at[kb + p + NBUF]], rows[u], gsem[u])

  pl.loop(0, NSUP64)(super_body)

  plsc.subcore_barrier()
  for m in range(NSTRIPE):
    pltpu.sync_copy(agg_sh.at[pl.ds(rbase + m * 128, 128)], r0)
    pltpu.sync_copy(r0, agg_out.at[pl.ds(c * NP + rbase + m * 128, 128)])


_sc_agg = pl.kernel(
    _sc_agg_body,
    out_type=jax.ShapeDtypeStruct((2 * NP, HALF), jnp.float32),
    mesh=_MESH,
    scratch_types=[
        pltpu.VMEM((C64_PER_TILE, CH), jnp.int32),
        pltpu.VMEM((SUP64, CH), jnp.int32),
        pltpu.VMEM((CH, HALF), jnp.float32),
        pltpu.VMEM((CH, HALF), jnp.float32),
        pltpu.VMEM_SHARED((NP, HALF), jnp.float32),
        pltpu.SemaphoreType.DMA,
        pltpu.SemaphoreType.DMA,
    ],
)


def _sc_deg_body(dst2, zrow, onesr, deg_out, didx, onesv, buf, deg_sh):
  c = lax.axis_index("c")
  s = lax.axis_index("s")
  rbase = s * ROWS_PER_TILE

  pltpu.sync_copy(onesr.at[pl.ds(0, CH)], onesv)
  pltpu.sync_copy(zrow, buf)
  for m in range(NSTRIPE):
    pltpu.sync_copy(buf, deg_sh.at[pl.ds(rbase + m * 128, 128)])
  plsc.subcore_barrier()

  chalf = C64 // 2
  cb0 = c * chalf + s * (chalf // 16)

  def super_body(k):
    pltpu.sync_copy(dst2.at[pl.ds(cb0 + k * SUP64, SUP64)], didx)
    for j in range(SUP64):
      pltpu.sync_copy(onesv, deg_sh.at[didx.at[j]], add=True)

  pl.loop(0, chalf // 16 // SUP64)(super_body)

  plsc.subcore_barrier()
  for m in range(NSTRIPE):
    pltpu.sync_copy(deg_sh.at[pl.ds(rbase + m * 128, 128)], buf)
    pltpu.sync_copy(buf, deg_out.at[pl.ds(c * NP + rbase + m * 128, 128)])


_sc_deg = pl.kernel(
    _sc_deg_body,
    out_type=jax.ShapeDtypeStruct((2 * NP, HALF), jnp.float32),
    mesh=_MESH,
    scratch_types=[
        pltpu.VMEM((SUP64, CH), jnp.int32),
        pltpu.VMEM((CH, HALF), jnp.float32),
        pltpu.VMEM((128, HALF), jnp.float32),
        pltpu.VMEM_SHARED((NP, HALF), jnp.float32),
    ],
)


def _t1_body(agg_ref, deg_ref, x_ref, wl0_ref, wl1_ref, wr_ref, b_ref,
             out_ref):
  r = 1.0 / jnp.maximum(deg_ref[0] + deg_ref[1], 1.0)
  a0 = agg_ref[0] * r
  a1 = agg_ref[1] * r
  h = jnp.dot(a0, wl0_ref[...], preferred_element_type=jnp.float32)
  h = h + jnp.dot(a1, wl1_ref[...], preferred_element_type=jnp.float32)
  h = h + jnp.dot(x_ref[...], wr_ref[...], preferred_element_type=jnp.float32)
  h = h + b_ref[0:1]
  h = jnp.maximum(h, 0.0)
  out_ref[0] = h[:, :HALF]
  out_ref[1] = h[:, HALF:]


_t1 = pl.pallas_call(
    _t1_body,
    grid=(NB,),
    in_specs=[
        pl.BlockSpec((2, B, HALF), lambda i: (0, i, 0)),
        pl.BlockSpec((2, B, HALF), lambda i: (0, i, 0)),
        pl.BlockSpec((B, D), lambda i: (i, 0)),
        pl.BlockSpec((HALF, D), lambda i: (0, 0)),
        pl.BlockSpec((HALF, D), lambda i: (0, 0)),
        pl.BlockSpec((D, D), lambda i: (0, 0)),
        pl.BlockSpec((8, D), lambda i: (0, 0)),
    ],
    out_specs=pl.BlockSpec((2, B, HALF), lambda i: (0, i, 0)),
    out_shape=jax.ShapeDtypeStruct((2, NP, HALF), jnp.float32),
)


def _t2_body(agg_ref, deg_ref, h1_ref, wl0_ref, wl1_ref, wr0_ref, wr1_ref,
             b_ref, batch_ref, c1w_ref, c1b_ref, c2w_ref, c2b_ref, c3w_ref,
             c3b_ref, out_ref, psum, cnt):
  i = pl.program_id(0)

  @pl.when(i == 0)
  def _():
    psum[...] = jnp.zeros_like(psum)
    cnt[...] = jnp.zeros_like(cnt)

  r = 1.0 / jnp.maximum(deg_ref[0] + deg_ref[1], 1.0)
  a0 = agg_ref[0] * r
  a1 = agg_ref[1] * r
  h = jnp.dot(a0, wl0_ref[...], preferred_element_type=jnp.float32)
  h = h + jnp.dot(a1, wl1_ref[...], preferred_element_type=jnp.float32)
  h = h + jnp.dot(h1_ref[0], wr0_ref[...], preferred_element_type=jnp.float32)
  h = h + jnp.dot(h1_ref[1], wr1_ref[...], preferred_element_type=jnp.float32)
  h = h + b_ref[0:1]
  h2 = jnp.maximum(h, 0.0)

  b = batch_ref[0]
  gid = lax.broadcasted_iota(jnp.int32, (G, B), 0)
  rowid = lax.broadcasted_iota(jnp.int32, (G, B), 1) + i * B
  mask = jnp.where((b == gid) & (rowid < N), 1.0, 0.0)
  psum[...] += jnp.dot(mask, h2, preferred_element_type=jnp.float32)
  cnt[...] += jnp.sum(mask, axis=1, keepdims=True)

  @pl.when(i == NB - 1)
  def _():
    cw = cnt[:, 0:1]
    pooled = psum[...] / jnp.maximum(cw, 1.0)
    z = jnp.dot(pooled, c1w_ref[...], preferred_element_type=jnp.float32)
    z = jnp.maximum(z + c1b_ref[0:1], 0.0)
    z = jnp.dot(z, c2w_ref[...], preferred_element_type=jnp.float32)
    z = jnp.maximum(z + c2b_ref[0:1], 0.0)
    o = jnp.dot(z, c3w_ref[...], preferred_element_type=jnp.float32)
    out_ref[...] = o + c3b_ref[0:1]


_t2 = pl.pallas_call(
    _t2_body,
    grid=(NB,),
    in_specs=[
        pl.BlockSpec((2, B, HALF), lambda i: (0, i, 0)),
        pl.BlockSpec((2, B, HALF), lambda i: (0, i, 0)),
        pl.BlockSpec((2, B, HALF), lambda i: (0, i, 0)),
        pl.BlockSpec((HALF, D), lambda i: (0, 0)),
        pl.BlockSpec((HALF, D), lambda i: (0, 0)),
        pl.BlockSpec((HALF, D), lambda i: (0, 0)),
        pl.BlockSpec((HALF, D), lambda i: (0, 0)),
        pl.BlockSpec((8, D), lambda i: (0, 0)),
        pl.BlockSpec((1, 1, B), lambda i: (i, 0, 0)),
        pl.BlockSpec((D, HALF), lambda i: (0, 0)),
        pl.BlockSpec((8, HALF), lambda i: (0, 0)),
        pl.BlockSpec((HALF, HALF), lambda i: (0, 0)),
        pl.BlockSpec((8, HALF), lambda i: (0, 0)),
        pl.BlockSpec((HALF, HALF), lambda i: (0, 0)),
        pl.BlockSpec((8, HALF), lambda i: (0, 0)),
    ],
    out_specs=pl.BlockSpec((G, HALF), lambda i: (0, 0)),
    out_shape=jax.ShapeDtypeStruct((G, HALF), jnp.float32),
    scratch_shapes=[
        pltpu.VMEM((G, D), jnp.float32),
        pltpu.VMEM((G, HALF), jnp.float32),
    ],
)


def kernel(x, edge_index, batch, Wl1, bl1, Wr1, Wl2, bl2, Wr2,
           C1W, C1b, C2W, C2b, C3W, C3b):
  f32 = jnp.float32
  src = edge_index[0].astype(jnp.int32)
  dst = edge_index[1].astype(jnp.int32)

  pad_e = EP - E
  srcp = jnp.concatenate([src, jnp.zeros((pad_e,), jnp.int32)])
  srcx = jnp.concatenate([srcp, srcp + NP]).reshape(2 * C64, CH)
  dst2 = jnp.concatenate([dst, jnp.full((pad_e,), N, jnp.int32)]).reshape(C64, CH)

  xp = jnp.pad(x, ((0, NP - N), (0, 0)))
  x2 = jnp.concatenate([xp[:, :HALF], xp[:, HALF:]], axis=0)

  zrow = jnp.zeros((128, HALF), f32)
  onesr = jnp.ones((128, HALF), f32)

  deg = _sc_deg(dst2, zrow, onesr).reshape(2, NP, HALF)
  agg1 = _sc_agg(x2, srcx, dst2, zrow).reshape(2, NP, HALF)

  wl1t = Wl1.T
  bl1b = jnp.broadcast_to(bl1[None, :], (8, D))
  h1 = _t1(agg1, deg, xp, wl1t[:HALF], wl1t[HALF:], Wr1.T, bl1b)

  agg2 = _sc_agg(h1.reshape(2 * NP, HALF), srcx, dst2, zrow).reshape(2, NP, HALF)

  wl2t = Wl2.T
  wr2t = Wr2.T
  bl2b = jnp.broadcast_to(bl2[None, :], (8, D))
  batch3 = jnp.concatenate([batch.astype(jnp.int32),
                            jnp.full((NP - N,), G, jnp.int32)]).reshape(NB, 1, B)
  c1wt = C1W.T
  c1bb = jnp.broadcast_to(C1b[None, :], (8, HALF))
  c2wt = jnp.zeros((HALF, HALF), f32).at[:, :64].set(C2W.T)
  c2bb = jnp.broadcast_to(jnp.zeros((HALF,), f32).at[:64].set(C2b)[None, :], (8, HALF))
  c3wt = jnp.zeros((HALF, HALF), f32).at[:64, 0].set(C3W[0])
  c3bb = jnp.broadcast_to(jnp.zeros((HALF,), f32).at[0].set(C3b[0])[None, :], (8, HALF))

  out128 = _t2(agg2, deg, h1, wl2t[:HALF], wl2t[HALF:],
               wr2t[:HALF], wr2t[HALF:], bl2b, batch3,
               c1wt, c1bb, c2wt, c2bb, c3wt, c3bb)
  return out128[:, :1]
